# baseline (speedup 1.0000x reference)
import functools

import jax
import jax.numpy as jnp
from jax import lax
from jax.experimental import pallas as pl
from jax.experimental.pallas import tpu as pltpu
from jax.experimental.pallas import tpu_sc as plsc

NUM_CORES = 2
NUM_SUBCORES = 16
NUM_WORKERS = NUM_CORES * NUM_SUBCORES

S = 26
B = 16384
D = 32
V = 1_000_000

BCH = 1024
NC_B = B // BCH
UNITS = S * NC_B
UPW = UNITS // NUM_WORKERS
GRP = 8
NGRP = D // GRP
STAGES = UPW * NGRP


def _make_gather():
  mesh = plsc.VectorSubcoreMesh(core_axis_name="c", subcore_axis_name="s")

  @functools.partial(
      pl.kernel,
      out_type=jax.ShapeDtypeStruct((S, D, B), jnp.float32),
      mesh=mesh,
      compiler_params=pltpu.CompilerParams(use_tc_tiling_on_sc=False),
      scratch_types=[
          pltpu.VMEM((2, BCH), jnp.int32),
          pltpu.VMEM((2, GRP, BCH), jnp.float32),
          pltpu.SemaphoreType.DMA,
          pltpu.SemaphoreType.DMA,
          pltpu.SemaphoreType.DMA,
      ],
  )
  def gather_kernel(idxT_hbm, wT_hbm, outT_hbm, idx_v, data_v, gsem, ssem, isem):
    wid = lax.axis_index("s") * NUM_CORES + lax.axis_index("c")
    u0 = wid * UPW

    def load_idx(uid, ib, sem):
      s = uid // NC_B
      c = uid % NC_B
      return pltpu.async_copy(
          idxT_hbm.at[s, pl.ds(pl.multiple_of(c * BCH, 8), BCH)],
          idx_v.at[ib], sem)

    load_idx(u0, 0, isem).wait()

    def stage(k, carry):
      uid = u0 + k // NGRP
      g = lax.rem(k, NGRP)
      s = uid // NC_B
      c = lax.rem(uid, NC_B)
      b = lax.rem(k, 2)
      ib = lax.rem(k // NGRP, 2)

      @pl.when(jnp.logical_and(g == 0, k > 0))
      def _():
        pltpu.make_async_copy(
            idxT_hbm.at[0, pl.ds(0, BCH)], idx_v.at[ib], isem).wait()

      @pl.when(k >= 2)
      def _():
        pltpu.make_async_copy(
            wT_hbm.at[pl.ds(0, GRP), pl.ds(0, BCH)], data_v.at[b], ssem).wait()

      handles = []
      for dd in range(GRP):
        d = g * GRP + dd
        handles.append(pltpu.async_copy(
            wT_hbm.at[d].at[idx_v.at[ib]], data_v.at[b, dd], gsem))

      @pl.when(jnp.logical_and(g == NGRP - 1, k < STAGES - NGRP))
      def _():
        load_idx(uid + 1, 1 - ib, isem)

      for h in handles:
        h.wait()

      for dd in range(GRP):
        d = g * GRP + dd
        pltpu.async_copy(
            data_v.at[b, dd],
            outT_hbm.at[s, d, pl.ds(pl.multiple_of(c * BCH, 8), BCH)],
            ssem)
      return carry

    lax.fori_loop(0, STAGES, stage, 0)

    for _ in range(2):
      pltpu.make_async_copy(
          wT_hbm.at[pl.ds(0, GRP), pl.ds(0, BCH)], data_v.at[0], ssem).wait()

  return gather_kernel


_gather = _make_gather()


def kernel(indices, weight):
  idxT = indices.T.astype(jnp.int32)
  wT = weight.T
  outT = _gather(idxT, wT)
  return outT.transpose(2, 0, 1)


# speedup vs baseline: 3.7437x; 3.7437x over previous
import functools

import jax
import jax.numpy as jnp
from jax import lax
from jax.experimental import pallas as pl
from jax.experimental.pallas import tpu as pltpu
from jax.experimental.pallas import tpu_sc as plsc

NUM_CORES = 2
NUM_SUBCORES = 16
NUM_WORKERS = NUM_CORES * NUM_SUBCORES

B_TOTAL = 16384 * 26
EMB_DIM = 32
V = 1_000_000

VB = 1664
T_ROWS = VB // 4
T_GRID = -(-V // VB)
V_PAD = T_GRID * VB
OUT_ROWS = T_GRID * T_ROWS

B_PER_W = B_TOTAL // NUM_WORKERS
CHUNK = 832
NCHUNK = B_PER_W // CHUNK
NBUF = 4


def _transpose_table(wT):
  def body(w_ref, o_ref):
    x = w_ref[...]
    y = jnp.swapaxes(x, 0, 1)
    z = y.reshape(T_ROWS, 4, EMB_DIM)
    o_ref[...] = jnp.concatenate([z[:, a, :] for a in range(4)], axis=-1)

  return pl.pallas_call(
      body,
      grid=(T_GRID,),
      in_specs=[pl.BlockSpec((EMB_DIM, VB), lambda i: (0, i))],
      out_specs=pl.BlockSpec((T_ROWS, 128), lambda i: (i, 0)),
      out_shape=jax.ShapeDtypeStruct((OUT_ROWS, 128), jnp.float32),
  )(wT)


def _make_gather():
  mesh = plsc.VectorSubcoreMesh(core_axis_name="c", subcore_axis_name="s")

  @functools.partial(
      pl.kernel,
      out_type=jax.ShapeDtypeStruct((B_TOTAL, EMB_DIM), jnp.float32),
      mesh=mesh,
      compiler_params=pltpu.CompilerParams(use_tc_tiling_on_sc=False),
      scratch_types=[
          pltpu.VMEM((B_PER_W,), jnp.int32),
          pltpu.VMEM((NBUF, CHUNK, EMB_DIM), jnp.float32),
      ] + [pltpu.SemaphoreType.DMA] * (2 * NBUF),
  )
  def gather_kernel(idx_hbm, table_hbm, out_hbm, idx_v, rows_v, *sems):
    gsem, ssem = sems[:NBUF], sems[NBUF:]
    wid = lax.axis_index("s") * NUM_CORES + lax.axis_index("c")
    base = pl.multiple_of(wid * B_PER_W, 8)
    pltpu.sync_copy(idx_hbm.at[pl.ds(base, B_PER_W)], idx_v)

    def start_gather(i, b):
      return pltpu.async_copy(
          table_hbm.at[idx_v.at[pl.ds(i * CHUNK, CHUNK)]],
          rows_v.at[b], gsem[b])

    def start_store(i, b):
      return pltpu.async_copy(
          rows_v.at[b],
          out_hbm.at[pl.ds(pl.multiple_of(base + i * CHUNK, 8), CHUNK)],
          ssem[b])

    gh = [start_gather(b, b) for b in range(NBUF)]
    sh = [None] * NBUF
    for i in range(NCHUNK):
      b = i % NBUF
      gh[b].wait()
      sh[b] = start_store(i, b)
      j = i + NBUF
      if j < NCHUNK:
        sh[b].wait()
        gh[b] = start_gather(j, b)
    for i in range(NCHUNK - NBUF, NCHUNK):
      sh[i % NBUF].wait()

  return gather_kernel


_gather = _make_gather()


def kernel(indices, weight):
  idx_flat = indices.reshape(-1).astype(jnp.int32)
  w128 = _transpose_table(weight.T)
  w_rows = w128.reshape(V_PAD, EMB_DIM)
  out = _gather(idx_flat, w_rows)
  return out.reshape(indices.shape[0], indices.shape[1], weight.shape[1])


# speedup vs baseline: 4.9731x; 1.3284x over previous
import functools

import jax
import jax.numpy as jnp
from jax import lax
from jax.experimental import pallas as pl
from jax.experimental.pallas import tpu as pltpu
from jax.experimental.pallas import tpu_sc as plsc

NUM_CORES = 2
NUM_SUBCORES = 16
NUM_WORKERS = NUM_CORES * NUM_SUBCORES

B_TOTAL = 16384 * 26
EMB_DIM = 32
V = 1_000_000

VB = 2048
VB4 = VB // 4
T_GRID = -(-V // VB)
V_PAD = T_GRID * VB
OUT_ROWS = T_GRID * VB4

B_PER_W = B_TOTAL // NUM_WORKERS
CHUNK = 832
NCHUNK = B_PER_W // CHUNK
NBUF = 4


def _transpose_table(wT):
  def body(w_ref, o_ref):
    x = w_ref[...]
    xq = jnp.concatenate(
        [x[:, q * VB4:(q + 1) * VB4] for q in range(4)], axis=0)
    o_ref[...] = jnp.swapaxes(xq, 0, 1)

  return pl.pallas_call(
      body,
      grid=(T_GRID,),
      in_specs=[pl.BlockSpec((EMB_DIM, VB), lambda i: (0, i))],
      out_specs=pl.BlockSpec((VB4, 128), lambda i: (i, 0)),
      out_shape=jax.ShapeDtypeStruct((OUT_ROWS, 128), jnp.float32),
  )(wT)


def _make_gather():
  mesh = plsc.VectorSubcoreMesh(core_axis_name="c", subcore_axis_name="s")

  @functools.partial(
      pl.kernel,
      out_type=jax.ShapeDtypeStruct((B_TOTAL, EMB_DIM), jnp.float32),
      mesh=mesh,
      compiler_params=pltpu.CompilerParams(use_tc_tiling_on_sc=False),
      scratch_types=[
          pltpu.VMEM((B_PER_W,), jnp.int32),
          pltpu.VMEM((NBUF, CHUNK, EMB_DIM), jnp.float32),
      ] + [pltpu.SemaphoreType.DMA] * (2 * NBUF),
  )
  def gather_kernel(idx_hbm, table_hbm, out_hbm, idx_v, rows_v, *sems):
    gsem, ssem = sems[:NBUF], sems[NBUF:]
    wid = lax.axis_index("s") * NUM_CORES + lax.axis_index("c")
    base = pl.multiple_of(wid * B_PER_W, 8)
    pltpu.sync_copy(idx_hbm.at[pl.ds(base, B_PER_W)], idx_v)

    def start_gather(i, b):
      return pltpu.async_copy(
          table_hbm.at[idx_v.at[pl.ds(i * CHUNK, CHUNK)]],
          rows_v.at[b], gsem[b])

    def start_store(i, b):
      return pltpu.async_copy(
          rows_v.at[b],
          out_hbm.at[pl.ds(pl.multiple_of(base + i * CHUNK, 8), CHUNK)],
          ssem[b])

    gh = [start_gather(b, b) for b in range(NBUF)]
    sh = [None] * NBUF
    for i in range(NCHUNK):
      b = i % NBUF
      gh[b].wait()
      sh[b] = start_store(i, b)
      j = i + NBUF
      if j < NCHUNK:
        sh[b].wait()
        gh[b] = start_gather(j, b)
    for i in range(NCHUNK - NBUF, NCHUNK):
      sh[i % NBUF].wait()

  return gather_kernel


_gather = _make_gather()


def kernel(indices, weight):
  idx = indices.reshape(-1).astype(jnp.int32)
  idx2 = ((((idx >> 11) << 9) + (idx & 511)) << 2) + ((idx >> 9) & 3)
  w128 = _transpose_table(weight.T)
  w_rows = w128.reshape(V_PAD, EMB_DIM)
  out = _gather(idx2, w_rows)
  return out.reshape(indices.shape[0], indices.shape[1], weight.shape[1])


# speedup vs baseline: 7.0205x; 1.4117x over previous
import functools

import jax
import jax.numpy as jnp
from jax import lax
from jax.experimental import pallas as pl
from jax.experimental.pallas import tpu as pltpu
from jax.experimental.pallas import tpu_sc as plsc

NUM_CORES = 2
NUM_SUBCORES = 16
NUM_WORKERS = NUM_CORES * NUM_SUBCORES

B_TOTAL = 16384 * 26
EMB_DIM = 32
V = 1_000_000

VB = 8192
VB4 = VB // 4
T_GRID = -(-V // VB)
V_PAD = T_GRID * VB
OUT_ROWS = T_GRID * VB4

B_PER_W = B_TOTAL // NUM_WORKERS
CHUNK = 832
NCHUNK = B_PER_W // CHUNK
NBUF = 4


def _transpose_table(wT):
  def body(w_ref, o_ref):
    x = w_ref[...]
    xq = jnp.concatenate(
        [x[:, q * VB4:(q + 1) * VB4] for q in range(4)], axis=0)
    o_ref[...] = jnp.swapaxes(xq, 0, 1)

  return pl.pallas_call(
      body,
      grid=(T_GRID,),
      in_specs=[pl.BlockSpec((EMB_DIM, VB), lambda i: (0, i))],
      out_specs=pl.BlockSpec((VB4, 128), lambda i: (i, 0)),
      out_shape=jax.ShapeDtypeStruct((OUT_ROWS, 128), jnp.float32),
  )(wT)


def _make_gather():
  mesh = plsc.VectorSubcoreMesh(core_axis_name="c", subcore_axis_name="s")

  @functools.partial(
      pl.kernel,
      out_type=jax.ShapeDtypeStruct((B_TOTAL, EMB_DIM), jnp.float32),
      mesh=mesh,
      compiler_params=pltpu.CompilerParams(use_tc_tiling_on_sc=False),
      scratch_types=[
          pltpu.VMEM((B_PER_W,), jnp.int32),
          pltpu.VMEM((NBUF, CHUNK, EMB_DIM), jnp.float32),
      ] + [pltpu.SemaphoreType.DMA] * (2 * NBUF),
  )
  def gather_kernel(idx_hbm, table_hbm, out_hbm, idx_v, rows_v, *sems):
    gsem, ssem = sems[:NBUF], sems[NBUF:]
    wid = lax.axis_index("s") * NUM_CORES + lax.axis_index("c")
    base = pl.multiple_of(wid * B_PER_W, 8)
    pltpu.sync_copy(idx_hbm.at[pl.ds(base, B_PER_W)], idx_v)

    def start_gather(i, b):
      return pltpu.async_copy(
          table_hbm.at[idx_v.at[pl.ds(i * CHUNK, CHUNK)]],
          rows_v.at[b], gsem[b])

    def start_store(i, b):
      return pltpu.async_copy(
          rows_v.at[b],
          out_hbm.at[pl.ds(pl.multiple_of(base + i * CHUNK, 8), CHUNK)],
          ssem[b])

    gh = [start_gather(b, b) for b in range(NBUF)]
    sh = [None] * NBUF
    for i in range(NCHUNK):
      b = i % NBUF
      gh[b].wait()
      sh[b] = start_store(i, b)
      j = i + NBUF
      if j < NCHUNK:
        sh[b].wait()
        gh[b] = start_gather(j, b)
    for i in range(NCHUNK - NBUF, NCHUNK):
      sh[i % NBUF].wait()

  return gather_kernel


_gather = _make_gather()


def kernel(indices, weight):
  idx = indices.reshape(-1).astype(jnp.int32)
  idx2 = ((((idx >> 13) << 11) + (idx & 2047)) << 2) + ((idx >> 11) & 3)
  w128 = _transpose_table(weight.T)
  w_rows = w128.reshape(V_PAD, EMB_DIM)
  out = _gather(idx2, w_rows)
  return out.reshape(indices.shape[0], indices.shape[1], weight.shape[1])


# speedup vs baseline: 7.8955x; 1.1246x over previous
import functools

import jax
import jax.numpy as jnp
from jax import lax
from jax.experimental import pallas as pl
from jax.experimental.pallas import tpu as pltpu
from jax.experimental.pallas import tpu_sc as plsc

NUM_CORES = 2
NUM_SUBCORES = 16
NUM_WORKERS = NUM_CORES * NUM_SUBCORES

B_TOTAL = 16384 * 26
EMB_DIM = 32
V = 1_000_000

VB = 32768
VB4 = VB // 4
T_GRID = -(-V // VB)
V_PAD = T_GRID * VB
OUT_ROWS = T_GRID * VB4

B_PER_W = B_TOTAL // NUM_WORKERS
CHUNK = 832
NCHUNK = B_PER_W // CHUNK
NBUF = 4


def _transpose_table(wT):
  def body(w_ref, o_ref):
    x = w_ref[...]
    xq = jnp.concatenate(
        [x[:, q * VB4:(q + 1) * VB4] for q in range(4)], axis=0)
    o_ref[...] = jnp.swapaxes(xq, 0, 1)

  return pl.pallas_call(
      body,
      grid=(T_GRID,),
      in_specs=[pl.BlockSpec((EMB_DIM, VB), lambda i: (0, i))],
      out_specs=pl.BlockSpec((VB4, 128), lambda i: (i, 0)),
      out_shape=jax.ShapeDtypeStruct((OUT_ROWS, 128), jnp.float32),
  )(wT)


def _make_gather():
  mesh = plsc.VectorSubcoreMesh(core_axis_name="c", subcore_axis_name="s")

  @functools.partial(
      pl.kernel,
      out_type=jax.ShapeDtypeStruct((B_TOTAL, EMB_DIM), jnp.float32),
      mesh=mesh,
      compiler_params=pltpu.CompilerParams(use_tc_tiling_on_sc=False),
      scratch_types=[
          pltpu.VMEM((B_PER_W,), jnp.int32),
          pltpu.VMEM((NBUF, CHUNK, EMB_DIM), jnp.float32),
      ] + [pltpu.SemaphoreType.DMA] * (2 * NBUF),
  )
  def gather_kernel(idx_hbm, table_hbm, out_hbm, idx_v, rows_v, *sems):
    gsem, ssem = sems[:NBUF], sems[NBUF:]
    wid = lax.axis_index("s") * NUM_CORES + lax.axis_index("c")
    base = pl.multiple_of(wid * B_PER_W, 8)
    pltpu.sync_copy(idx_hbm.at[pl.ds(base, B_PER_W)], idx_v)

    def start_gather(i, b):
      return pltpu.async_copy(
          table_hbm.at[idx_v.at[pl.ds(i * CHUNK, CHUNK)]],
          rows_v.at[b], gsem[b])

    def start_store(i, b):
      return pltpu.async_copy(
          rows_v.at[b],
          out_hbm.at[pl.ds(pl.multiple_of(base + i * CHUNK, 8), CHUNK)],
          ssem[b])

    gh = [start_gather(b, b) for b in range(NBUF)]
    sh = [None] * NBUF
    for i in range(NCHUNK):
      b = i % NBUF
      gh[b].wait()
      sh[b] = start_store(i, b)
      j = i + NBUF
      if j < NCHUNK:
        sh[b].wait()
        gh[b] = start_gather(j, b)
    for i in range(NCHUNK - NBUF, NCHUNK):
      sh[i % NBUF].wait()

  return gather_kernel


_gather = _make_gather()


def kernel(indices, weight):
  idx = indices.reshape(-1).astype(jnp.int32)
  idx2 = ((((idx >> 15) << 13) + (idx & 8191)) << 2) + ((idx >> 13) & 3)
  w128 = _transpose_table(weight.T)
  w_rows = w128.reshape(V_PAD, EMB_DIM)
  out = _gather(idx2, w_rows)
  return out.reshape(indices.shape[0], indices.shape[1], weight.shape[1])


# speedup vs baseline: 13.8508x; 1.7543x over previous
import functools

import jax
import jax.numpy as jnp
from jax import lax
from jax.experimental import pallas as pl
from jax.experimental.pallas import tpu as pltpu
from jax.experimental.pallas import tpu_sc as plsc

NUM_CORES = 2
NUM_SUBCORES = 16
NUM_WORKERS = NUM_CORES * NUM_SUBCORES

S = 26
B = 16384
B_TOTAL = B * S
EMB_DIM = 32
V = 1_000_000

VB = 32768
VB4 = VB // 4
T_GRID = -(-V // VB)
V_PAD = T_GRID * VB
OUT_ROWS = T_GRID * VB4

B_PER_W = B_TOTAL // NUM_WORKERS
CHUNK = 1024
NCHUNK = B_PER_W // CHUNK
NBUF = 3
NR = B // 4

BR = 1024
NBLK = NR // BR


def _transpose_table(wT):
  def body(w_ref, o_ref):
    x = w_ref[...]
    xq = jnp.concatenate(
        [x[:, q * VB4:(q + 1) * VB4] for q in range(4)], axis=0)
    o_ref[...] = jnp.swapaxes(xq, 0, 1)

  return pl.pallas_call(
      body,
      grid=(T_GRID,),
      in_specs=[pl.BlockSpec((EMB_DIM, VB), lambda i: (0, i))],
      out_specs=pl.BlockSpec((VB4, 128), lambda i: (i, 0)),
      out_shape=jax.ShapeDtypeStruct((OUT_ROWS, 128), jnp.float32),
  )(wT)


def _untranspose_out(n128):
  def body(i_ref, o_ref):
    y = i_ref[...]
    z = jnp.swapaxes(y, 0, 1)
    o_ref[...] = jnp.concatenate(
        [z[32 * a:32 * (a + 1), :] for a in range(4)], axis=1)[None]

  return pl.pallas_call(
      body,
      grid=(S, NBLK),
      in_specs=[pl.BlockSpec((BR, 128), lambda s, blk: (s * NBLK + blk, 0))],
      out_specs=pl.BlockSpec((1, EMB_DIM, 4 * BR), lambda s, blk: (s, 0, blk)),
      out_shape=jax.ShapeDtypeStruct((S, EMB_DIM, B), jnp.float32),
  )(n128)


def _make_gather():
  mesh = plsc.VectorSubcoreMesh(core_axis_name="c", subcore_axis_name="s")

  @functools.partial(
      pl.kernel,
      out_type=jax.ShapeDtypeStruct((S, NR, 128), jnp.float32),
      mesh=mesh,
      compiler_params=pltpu.CompilerParams(use_tc_tiling_on_sc=False),
      scratch_types=[
          pltpu.VMEM((B_PER_W,), jnp.int32),
          pltpu.VMEM((NBUF, CHUNK, EMB_DIM), jnp.float32),
      ] + [pltpu.SemaphoreType.DMA] * (2 * NBUF),
  )
  def gather_kernel(idx_hbm, table_hbm, out_hbm, idx_v, rows_v, *sems):
    gsem, ssem = sems[:NBUF], sems[NBUF:]
    wid = lax.axis_index("s") * NUM_CORES + lax.axis_index("c")
    base = pl.multiple_of(wid * B_PER_W, 8)
    pltpu.sync_copy(idx_hbm.at[pl.ds(base, B_PER_W)], idx_v)

    def start_gather(i, b):
      return pltpu.async_copy(
          table_hbm.at[idx_v.at[pl.ds(i * CHUNK, CHUNK)]],
          rows_v.at[b], gsem[b])

    def start_store(i, b):
      uid = wid * NCHUNK + i
      s = uid >> 4
      blk = (uid >> 2) & 3
      a = uid & 3
      return pltpu.async_copy(
          rows_v.at[b],
          out_hbm.at[s,
                     pl.ds(pl.multiple_of(blk * BR, 8), CHUNK),
                     pl.ds(pl.multiple_of(a * EMB_DIM, 8), EMB_DIM)],
          ssem[b])

    gh = [start_gather(b, b) for b in range(NBUF)]
    sh = [None] * NBUF
    for i in range(NCHUNK):
      b = i % NBUF
      gh[b].wait()
      sh[b] = start_store(i, b)
      j = i + NBUF
      if j < NCHUNK:
        sh[b].wait()
        gh[b] = start_gather(j, b)
    for i in range(NCHUNK - NBUF, NCHUNK):
      sh[i % NBUF].wait()

  return gather_kernel


_gather = _make_gather()


def kernel(indices, weight):
  idx = indices.astype(jnp.int32)
  idx2 = ((((idx >> 15) << 13) + (idx & 8191)) << 2) + ((idx >> 13) & 3)
  idx_sc = jnp.transpose(
      idx2.reshape(4, 4, BR, S), (3, 0, 1, 2)).reshape(-1)
  w128 = _transpose_table(weight.T)
  w_rows = w128.reshape(V_PAD, EMB_DIM)
  o4 = _gather(idx_sc, w_rows)
  out3 = _untranspose_out(o4.reshape(S * NR, 128))
  return jnp.transpose(out3, (2, 0, 1))


# speedup vs baseline: 16.8013x; 1.2130x over previous
import functools

import jax
import jax.numpy as jnp
from jax import lax
from jax.experimental import pallas as pl
from jax.experimental.pallas import tpu as pltpu
from jax.experimental.pallas import tpu_sc as plsc

NUM_CORES = 2
NUM_SUBCORES = 16
NUM_WORKERS = NUM_CORES * NUM_SUBCORES

S = 26
B = 16384
B_TOTAL = B * S
EMB_DIM = 32
V = 1_000_000

VB = 65536
VB4 = VB // 4
T_GRID = -(-V // VB)
V_PAD = T_GRID * VB
OUT_ROWS = T_GRID * VB4

B_PER_W = B_TOTAL // NUM_WORKERS
CHUNK = 1024
NCHUNK = B_PER_W // CHUNK
NBUF = 3
NR = B // 4

BR = 4096


def _transpose_table(wT):
  def body(w_ref, o_ref):
    x = w_ref[...]
    xq = jnp.concatenate(
        [x[:, q * VB4:(q + 1) * VB4] for q in range(4)], axis=0)
    o_ref[...] = jnp.swapaxes(xq, 0, 1)

  return pl.pallas_call(
      body,
      grid=(T_GRID,),
      in_specs=[pl.BlockSpec((EMB_DIM, VB), lambda i: (0, i))],
      out_specs=pl.BlockSpec((VB4, 128), lambda i: (i, 0)),
      out_shape=jax.ShapeDtypeStruct((OUT_ROWS, 128), jnp.float32),
  )(wT)


def _untranspose_out(n128):
  def body(i_ref, o_ref):
    y = i_ref[...]
    z = jnp.swapaxes(y, 0, 1)
    o_ref[...] = jnp.concatenate(
        [z[32 * a:32 * (a + 1), :] for a in range(4)], axis=1)[None]

  return pl.pallas_call(
      body,
      grid=(S,),
      in_specs=[pl.BlockSpec((BR, 128), lambda s: (s, 0))],
      out_specs=pl.BlockSpec((1, EMB_DIM, 4 * BR), lambda s: (s, 0, 0)),
      out_shape=jax.ShapeDtypeStruct((S, EMB_DIM, B), jnp.float32),
  )(n128)


def _make_gather():
  mesh = plsc.VectorSubcoreMesh(core_axis_name="c", subcore_axis_name="s")

  @functools.partial(
      pl.kernel,
      out_type=jax.ShapeDtypeStruct((S, NR, 128), jnp.float32),
      mesh=mesh,
      compiler_params=pltpu.CompilerParams(use_tc_tiling_on_sc=False),
      scratch_types=[
          pltpu.VMEM((B_PER_W,), jnp.int32),
          pltpu.VMEM((NBUF, CHUNK, EMB_DIM), jnp.float32),
      ] + [pltpu.SemaphoreType.DMA] * (2 * NBUF),
  )
  def gather_kernel(idx_hbm, table_hbm, out_hbm, idx_v, rows_v, *sems):
    gsem, ssem = sems[:NBUF], sems[NBUF:]
    wid = lax.axis_index("s") * NUM_CORES + lax.axis_index("c")
    base = pl.multiple_of(wid * B_PER_W, 8)
    pltpu.sync_copy(idx_hbm.at[pl.ds(base, B_PER_W)], idx_v)

    def start_gather(i, b):
      return pltpu.async_copy(
          table_hbm.at[idx_v.at[pl.ds(i * CHUNK, CHUNK)]],
          rows_v.at[b], gsem[b])

    def start_store(i, b):
      uid = wid * NCHUNK + i
      s = uid >> 4
      a = (uid >> 2) & 3
      c4 = uid & 3
      return pltpu.async_copy(
          rows_v.at[b],
          out_hbm.at[s,
                     pl.ds(pl.multiple_of(c4 * CHUNK, 8), CHUNK),
                     pl.ds(pl.multiple_of(a * EMB_DIM, 8), EMB_DIM)],
          ssem[b])

    gh = [start_gather(b, b) for b in range(NBUF)]
    sh = [None] * NBUF
    for i in range(NCHUNK):
      b = i % NBUF
      gh[b].wait()
      sh[b] = start_store(i, b)
      j = i + NBUF
      if j < NCHUNK:
        sh[b].wait()
        gh[b] = start_gather(j, b)
    for i in range(NCHUNK - NBUF, NCHUNK):
      sh[i % NBUF].wait()

  return gather_kernel


_gather = _make_gather()


def kernel(indices, weight):
  idx = indices.astype(jnp.int32)
  idx2 = ((((idx >> 16) << 14) + (idx & 16383)) << 2) + ((idx >> 14) & 3)
  idx_sc = jnp.transpose(idx2).reshape(-1)
  w128 = _transpose_table(weight.T)
  w_rows = w128.reshape(V_PAD, EMB_DIM)
  o4 = _gather(idx_sc, w_rows)
  out3 = _untranspose_out(o4.reshape(S * NR, 128))
  return jnp.transpose(out3, (2, 0, 1))


# speedup vs baseline: 17.4756x; 1.0401x over previous
import functools

import jax
import jax.numpy as jnp
from jax import lax
from jax.experimental import pallas as pl
from jax.experimental.pallas import tpu as pltpu
from jax.experimental.pallas import tpu_sc as plsc

NUM_CORES = 2
NUM_SUBCORES = 16
NUM_WORKERS = NUM_CORES * NUM_SUBCORES

S = 26
B = 16384
B_TOTAL = B * S
EMB_DIM = 32
V = 1_000_000

VB = 65536
VB4 = VB // 4
T_GRID = -(-V // VB)
V_PAD = T_GRID * VB
OUT_ROWS = T_GRID * VB4

B_PER_W = B_TOTAL // NUM_WORKERS
CHUNK = 512
NCHUNK = B_PER_W // CHUNK
NBUF = 6
NR = B // 4

BR = 4096


def _transpose_table(wT):
  def body(w_ref, o_ref):
    x = w_ref[...]
    xq = jnp.concatenate(
        [x[:, q * VB4:(q + 1) * VB4] for q in range(4)], axis=0)
    o_ref[...] = jnp.swapaxes(xq, 0, 1)

  return pl.pallas_call(
      body,
      grid=(T_GRID,),
      in_specs=[pl.BlockSpec((EMB_DIM, VB), lambda i: (0, i))],
      out_specs=pl.BlockSpec((VB4, 128), lambda i: (i, 0)),
      out_shape=jax.ShapeDtypeStruct((OUT_ROWS, 128), jnp.float32),
  )(wT)


def _untranspose_out(n128):
  def body(i_ref, o_ref):
    for h in range(2):
      y = i_ref[h * NR:(h + 1) * NR, :]
      z = jnp.swapaxes(y, 0, 1)
      o_ref[h] = jnp.concatenate(
          [z[32 * a:32 * (a + 1), :] for a in range(4)], axis=1)

  return pl.pallas_call(
      body,
      grid=(S // 2,),
      in_specs=[pl.BlockSpec((2 * NR, 128), lambda s: (s, 0))],
      out_specs=pl.BlockSpec((2, EMB_DIM, B), lambda s: (s, 0, 0)),
      out_shape=jax.ShapeDtypeStruct((S, EMB_DIM, B), jnp.float32),
  )(n128)


def _make_gather():
  mesh = plsc.VectorSubcoreMesh(core_axis_name="c", subcore_axis_name="s")

  @functools.partial(
      pl.kernel,
      out_type=jax.ShapeDtypeStruct((S, NR, 128), jnp.float32),
      mesh=mesh,
      compiler_params=pltpu.CompilerParams(use_tc_tiling_on_sc=False),
      scratch_types=[
          pltpu.VMEM((B_PER_W,), jnp.int32),
          pltpu.VMEM((NBUF, CHUNK, EMB_DIM), jnp.float32),
      ] + [pltpu.SemaphoreType.DMA] * (2 * NBUF),
  )
  def gather_kernel(idx_hbm, table_hbm, out_hbm, idx_v, rows_v, *sems):
    gsem, ssem = sems[:NBUF], sems[NBUF:]
    wid = lax.axis_index("s") * NUM_CORES + lax.axis_index("c")
    base = pl.multiple_of(wid * B_PER_W, 8)
    pltpu.sync_copy(idx_hbm.at[pl.ds(base, B_PER_W)], idx_v)

    def start_gather(i, b):
      return pltpu.async_copy(
          table_hbm.at[idx_v.at[pl.ds(i * CHUNK, CHUNK)]],
          rows_v.at[b], gsem[b])

    def start_store(i, b):
      uid = wid * NCHUNK + i
      s = uid >> 5
      a = (uid >> 3) & 3
      c4 = uid & 7
      return pltpu.async_copy(
          rows_v.at[b],
          out_hbm.at[s,
                     pl.ds(pl.multiple_of(c4 * CHUNK, 8), CHUNK),
                     pl.ds(pl.multiple_of(a * EMB_DIM, 8), EMB_DIM)],
          ssem[b])

    gh = [start_gather(b, b) for b in range(NBUF)]
    sh = [None] * NBUF
    for i in range(NCHUNK):
      b = i % NBUF
      gh[b].wait()
      sh[b] = start_store(i, b)
      j = i + NBUF
      if j < NCHUNK:
        sh[b].wait()
        gh[b] = start_gather(j, b)
    for i in range(NCHUNK - NBUF, NCHUNK):
      sh[i % NBUF].wait()

  return gather_kernel


_gather = _make_gather()


def kernel(indices, weight):
  idx = indices.astype(jnp.int32)
  idx2 = ((((idx >> 16) << 14) + (idx & 16383)) << 2) + ((idx >> 14) & 3)
  idx_sc = jnp.transpose(idx2).reshape(-1)
  w128 = _transpose_table(weight.T)
  w_rows = w128.reshape(V_PAD, EMB_DIM)
  o4 = _gather(idx_sc, w_rows)
  out3 = _untranspose_out(o4.reshape(S * NR, 128))
  return jnp.transpose(out3, (2, 0, 1))


# speedup vs baseline: 17.4940x; 1.0011x over previous
import functools

import jax
import jax.numpy as jnp
from jax import lax
from jax.experimental import pallas as pl
from jax.experimental.pallas import tpu as pltpu
from jax.experimental.pallas import tpu_sc as plsc

NUM_CORES = 2
NUM_SUBCORES = 16
NUM_WORKERS = NUM_CORES * NUM_SUBCORES

S = 26
B = 16384
B_TOTAL = B * S
EMB_DIM = 32
V = 1_000_000

VB = 65536
VB4 = VB // 4
T_GRID = -(-V // VB)
V_PAD = T_GRID * VB
OUT_ROWS = T_GRID * VB4

B_PER_W = B_TOTAL // NUM_WORKERS
CHUNK = 512
NCHUNK = B_PER_W // CHUNK
NBUF = 6
NR = B // 4


def _transpose_table(wT):
  def body(w_ref, o_ref):
    x = w_ref[...]
    xq = jnp.concatenate(
        [x[:, q * VB4:(q + 1) * VB4] for q in range(4)], axis=0)
    o_ref[...] = jnp.swapaxes(xq, 0, 1)

  return pl.pallas_call(
      body,
      grid=(T_GRID,),
      in_specs=[pl.BlockSpec((EMB_DIM, VB), lambda i: (0, i))],
      out_specs=pl.BlockSpec((VB4, 128), lambda i: (i, 0)),
      out_shape=jax.ShapeDtypeStruct((OUT_ROWS, 128), jnp.float32),
  )(wT)


def _untranspose_out(n128):
  def body(i_ref, o_ref):
    for h in range(2):
      y = i_ref[h * NR:(h + 1) * NR, :]
      z = jnp.swapaxes(y, 0, 1)
      o_ref[h] = jnp.concatenate(
          [z[32 * a:32 * (a + 1), :] for a in range(4)], axis=1)

  return pl.pallas_call(
      body,
      grid=(S // 2,),
      in_specs=[pl.BlockSpec((2 * NR, 128), lambda s: (s, 0))],
      out_specs=pl.BlockSpec((2, EMB_DIM, B), lambda s: (s, 0, 0)),
      out_shape=jax.ShapeDtypeStruct((S, EMB_DIM, B), jnp.float32),
  )(n128)


def _make_gather():
  mesh = plsc.VectorSubcoreMesh(core_axis_name="c", subcore_axis_name="s")

  @functools.partial(
      pl.kernel,
      out_type=jax.ShapeDtypeStruct((S, NR, 128), jnp.float32),
      mesh=mesh,
      compiler_params=pltpu.CompilerParams(use_tc_tiling_on_sc=False),
      scratch_types=[
          pltpu.VMEM((B_PER_W,), jnp.int32),
          pltpu.VMEM((NBUF, CHUNK, EMB_DIM), jnp.float32),
      ] + [pltpu.SemaphoreType.DMA] * (2 * NBUF),
  )
  def gather_kernel(idx_hbm, table_hbm, out_hbm, idx_v, rows_v, *sems):
    gsem, ssem = sems[:NBUF], sems[NBUF:]
    wid = lax.axis_index("s") * NUM_CORES + lax.axis_index("c")
    base = pl.multiple_of(wid * B_PER_W, 8)
    pltpu.sync_copy(idx_hbm.at[pl.ds(base, B_PER_W)], idx_v)

    def start_gather(i, b):
      return pltpu.async_copy(
          table_hbm.at[idx_v.at[pl.ds(i * CHUNK, CHUNK)]],
          rows_v.at[b], gsem[b])

    def start_store(i, b):
      uid = wid * NCHUNK + i
      s = uid >> 5
      a = (uid >> 3) & 3
      c4 = uid & 7
      return pltpu.async_copy(
          rows_v.at[b],
          out_hbm.at[s,
                     pl.ds(pl.multiple_of(c4 * CHUNK, 8), CHUNK),
                     pl.ds(pl.multiple_of(a * EMB_DIM, 8), EMB_DIM)],
          ssem[b])

    gh = [start_gather(b, b) for b in range(NBUF)]
    sh = [None] * NBUF
    for i in range(NCHUNK):
      b = i % NBUF
      gh[b].wait()
      sh[b] = start_store(i, b)
      j = i + NBUF
      if j < NCHUNK:
        sh[b].wait()
        gh[b] = start_gather(j, b)
    for i in range(NCHUNK - NBUF, NCHUNK):
      sh[i % NBUF].wait()

  return gather_kernel


_gather = _make_gather()


def kernel(indices, weight):
  idx = indices.astype(jnp.int32)
  idx2 = ((((idx >> 16) << 14) + (idx & 16383)) << 2) + ((idx >> 14) & 3)
  idx_sc = jnp.transpose(idx2).reshape(-1)
  w128 = _transpose_table(weight.T)
  w_rows = w128.reshape(V_PAD, EMB_DIM)
  o4 = _gather(idx_sc, w_rows)
  out3 = _untranspose_out(o4.reshape(S * NR, 128))
  return jnp.transpose(out3, (2, 0, 1))
